# baseline (speedup 1.0000x reference)
import functools

import jax
import jax.numpy as jnp
from jax import lax
from jax.experimental import pallas as pl
from jax.experimental.pallas import tpu as pltpu
from jax.experimental.pallas import tpu_sc as plsc

N = 10000
E = 320000
D = 128
ED = 16

NC = 2
NS = 16
NW = NC * NS
SC_BLK = 128
E_PAD = 327680
PER_W = E_PAD // NW
NBLK = PER_W // SC_BLK
ROWS_PER_SUB = 632
ACC_ROWS = NS * ROWS_PER_SUB

EBLK = 1280
N_REAL_BLOCKS = E // EBLK
NPAD = 10240

def _mesh():
    return plsc.VectorSubcoreMesh(core_axis_name="c", subcore_axis_name="s",
                                  num_cores=NC)


def _dgT(x, w):
    return lax.dot_general(x, w, (((1,), (1,)), ((), ())),
                           preferred_element_type=jnp.float32)


NBG = 2


def _sc_gather(table, idx):
    T, W = table.shape
    rows_per_sub = T // NS
    assert T % (16 * NS) == 0

    dt = table.dtype

    @functools.partial(
        pl.kernel,
        out_type=jax.ShapeDtypeStruct((E_PAD, W), dt),
        mesh=_mesh(),
        scratch_types=[
            pltpu.VMEM((NBG, SC_BLK), jnp.int32),
            pltpu.VMEM((NBG, SC_BLK, W), dt),
            pltpu.VMEM_SHARED((T, W), dt),
        ] + [pltpu.SemaphoreType.DMA] * (3 * NBG),
    )
    def k(table_hbm, idx_hbm, out_hbm, idxs, bufs, tab_sh, *sems):
        isem = sems[:NBG]
        gsem = sems[NBG:2 * NBG]
        ssem = sems[2 * NBG:]
        cid = lax.axis_index("c")
        sid = lax.axis_index("s")
        wid = sid * NC + cid
        base = pl.multiple_of(wid * PER_W, SC_BLK)

        r0 = pl.multiple_of(sid * rows_per_sub, 16)
        pltpu.sync_copy(table_hbm.at[pl.ds(r0, rows_per_sub)],
                        tab_sh.at[pl.ds(r0, rows_per_sub)])

        def start_idx(blk, b):
            off = pl.multiple_of(base + blk * SC_BLK, SC_BLK)
            pltpu.make_async_copy(idx_hbm.at[pl.ds(off, SC_BLK)],
                                  idxs.at[b], isem[b]).start()

        for b in range(NBG):
            start_idx(b, b)

        plsc.subcore_barrier()

        @pl.loop(0, NBLK, step=NBG)
        def _(i):
            for b in range(NBG):
                blk = i + b
                off = pl.multiple_of(base + blk * SC_BLK, SC_BLK)
                pltpu.make_async_copy(idx_hbm.at[pl.ds(off, SC_BLK)],
                                      idxs.at[b], isem[b]).wait()
                pltpu.make_async_copy(tab_sh.at[idxs.at[b]], bufs.at[b],
                                      gsem[b]).start()
            for b in range(NBG):
                pltpu.make_async_copy(tab_sh.at[idxs.at[b]], bufs.at[b],
                                      gsem[b]).wait()
                off = pl.multiple_of(base + (i + b) * SC_BLK, SC_BLK)
                pltpu.make_async_copy(bufs.at[b],
                                      out_hbm.at[pl.ds(off, SC_BLK)],
                                      ssem[b]).start()
            for b in range(NBG):
                blk = i + b
                off = pl.multiple_of(base + blk * SC_BLK, SC_BLK)
                pltpu.make_async_copy(bufs.at[b],
                                      out_hbm.at[pl.ds(off, SC_BLK)],
                                      ssem[b]).wait()
                nxt = blk + NBG
                nxt = jnp.where(nxt >= NBLK, nxt - NBLK, nxt)
                start_idx(nxt, b)

        for b in range(NBG):
            pltpu.make_async_copy(idx_hbm.at[pl.ds(base, SC_BLK)],
                                  idxs.at[b], isem[b]).wait()

    return k(table, idx)


def _sc_scatter_add(msg, dst, zeros):
    NBS = 2

    @functools.partial(
        pl.kernel,
        out_type=jax.ShapeDtypeStruct((NC, ACC_ROWS, D), jnp.float32),
        mesh=_mesh(),
        scratch_types=[
            pltpu.VMEM((NBS, SC_BLK), jnp.int32),
            pltpu.VMEM((NBS, SC_BLK, D), jnp.float32),
            pltpu.VMEM_SHARED((ACC_ROWS, D), jnp.float32),
        ] + [pltpu.SemaphoreType.DMA] * (2 * NBS),
    )
    def k(msg_hbm, dst_hbm, zero_hbm, out_hbm, idxs, bufs, acc_sh, *sems):
        isem, msem = sems[:NBS], sems[NBS:]
        cid = lax.axis_index("c")
        sid = lax.axis_index("s")
        zoff = pl.multiple_of(sid * ROWS_PER_SUB, 8)
        pltpu.sync_copy(zero_hbm.at[pl.ds(zoff, ROWS_PER_SUB)],
                        acc_sh.at[pl.ds(zoff, ROWS_PER_SUB)])

        base = pl.multiple_of(cid * (E_PAD // NC) + sid * PER_W, SC_BLK)

        def start_loads(blk, b):
            off = pl.multiple_of(base + blk * SC_BLK, SC_BLK)
            pltpu.make_async_copy(dst_hbm.at[pl.ds(off, SC_BLK)],
                                  idxs.at[b], isem[b]).start()
            pltpu.make_async_copy(msg_hbm.at[pl.ds(off, SC_BLK)],
                                  bufs.at[b], msem[b]).start()

        for b in range(NBS):
            start_loads(b, b)

        plsc.subcore_barrier()

        @pl.loop(0, NBLK, step=NBS)
        def _(i):
            for b in range(NBS):
                blk = i + b
                off = pl.multiple_of(base + blk * SC_BLK, SC_BLK)
                pltpu.make_async_copy(dst_hbm.at[pl.ds(off, SC_BLK)],
                                      idxs.at[b], isem[b]).wait()
                pltpu.make_async_copy(msg_hbm.at[pl.ds(off, SC_BLK)],
                                      bufs.at[b], msem[b]).wait()
                pltpu.sync_copy(bufs.at[b], acc_sh.at[idxs.at[b]], add=True)
                nxt = blk + NBS
                nxt = jnp.where(nxt >= NBLK, nxt - NBLK, nxt)
                start_loads(nxt, b)

        for b in range(NBS):
            off = pl.multiple_of(base + b * SC_BLK, SC_BLK)
            pltpu.make_async_copy(dst_hbm.at[pl.ds(off, SC_BLK)],
                                  idxs.at[b], isem[b]).wait()
            pltpu.make_async_copy(msg_hbm.at[pl.ds(off, SC_BLK)],
                                  bufs.at[b], msem[b]).wait()

        plsc.subcore_barrier()
        pltpu.sync_copy(acc_sh.at[pl.ds(zoff, ROWS_PER_SUB)],
                        out_hbm.at[cid].at[pl.ds(zoff, ROWS_PER_SUB)])

    return k(msg, dst, zeros)


def _msg_kernel(g_ref, ef_ref, we_ref, be_ref, o_ref):
    g = g_ref[...].astype(jnp.float32)
    o_ref[...] = jnp.maximum(
        g + _dgT(ef_ref[...], we_ref[...]) + be_ref[...], 0.0)


def _tc_msg(G, efp, We, be2d):
    return pl.pallas_call(
        _msg_kernel,
        grid=(E_PAD // EBLK,),
        in_specs=[
            pl.BlockSpec((EBLK, D), lambda i: (i, 0)),
            pl.BlockSpec((EBLK, ED), lambda i: (i, 0)),
            pl.BlockSpec((D, ED), lambda i: (0, 0)),
            pl.BlockSpec((1, D), lambda i: (0, 0)),
        ],
        out_specs=pl.BlockSpec((EBLK, D), lambda i: (i, 0)),
        out_shape=jax.ShapeDtypeStruct((E_PAD, D), jnp.float32),
    )(G, efp, We, be2d)


def _node_kernel(em_ref, parts_ref, w1_ref, b1_ref, w2_ref, b2_ref,
                 gx_ref, bx_ref, wa_ref, wb_ref, xem_ref, p_ref, q_ref):
    h = em_ref[...] + parts_ref[0, :N, :] + parts_ref[1, :N, :]
    h = jnp.maximum(_dgT(h, w1_ref[...]) + b1_ref[...], 0.0)
    h = _dgT(h, w2_ref[...]) + b2_ref[...]
    mu = jnp.mean(h, axis=0, keepdims=True)
    var = jnp.mean((h - mu) ** 2, axis=0, keepdims=True)
    xem = (h - mu) * lax.rsqrt(var + 1e-5) * gx_ref[...] + bx_ref[...]
    xem = jnp.maximum(xem, 0.0)
    xem_ref[...] = xem
    p_ref[:N, :] = _dgT(xem, wa_ref[...]).astype(jnp.bfloat16)
    q_ref[:N, :] = _dgT(xem, wb_ref[...]).astype(jnp.bfloat16)


def _tc_node(em, parts, W1, b1r, W2, b2r, gxr, bxr, Wa, Wb):
    return pl.pallas_call(
        _node_kernel,
        out_shape=[
            jax.ShapeDtypeStruct((N, D), jnp.float32),
            jax.ShapeDtypeStruct((NPAD, D), jnp.bfloat16),
            jax.ShapeDtypeStruct((NPAD, D), jnp.bfloat16),
        ],
    )(em, parts, W1, b1r, W2, b2r, gxr, bxr, Wa, Wb)


def _edge12_kernel(gp_ref, gq_ref, ef_ref, wc_ref, bl1_ref, wl2_ref, bl2_ref,
                   f2_ref, msum_ref, c_ref):
    pid = pl.program_id(0)

    @pl.when(pid == 0)
    def _():
        msum_ref[...] = jnp.zeros_like(msum_ref)
        c_ref[...] = jnp.zeros_like(c_ref)

    gpq = gp_ref[...].astype(jnp.float32) + gq_ref[...].astype(jnp.float32)
    f1 = jnp.maximum(
        gpq + _dgT(ef_ref[...], wc_ref[...]) + bl1_ref[...], 0.0)
    f2 = jnp.maximum(_dgT(f1, wl2_ref[...]) + bl2_ref[...], 0.0)
    f2_ref[...] = f2.astype(jnp.bfloat16)

    @pl.when(pid < N_REAL_BLOCKS)
    def _():
        msum_ref[...] += jnp.sum(f2, axis=0, keepdims=True)
        c_ref[...] += lax.dot_general(f2, f2, (((0,), (0,)), ((), ())),
                                      preferred_element_type=jnp.float32)


def _tc_edge12(GP, GQ, efp, Wc, bl1r, Wl2, bl2r):
    return pl.pallas_call(
        _edge12_kernel,
        grid=(E_PAD // EBLK,),
        in_specs=[
            pl.BlockSpec((EBLK, D), lambda i: (i, 0)),
            pl.BlockSpec((EBLK, D), lambda i: (i, 0)),
            pl.BlockSpec((EBLK, ED), lambda i: (i, 0)),
            pl.BlockSpec((D, ED), lambda i: (0, 0)),
            pl.BlockSpec((1, D), lambda i: (0, 0)),
            pl.BlockSpec((D, D), lambda i: (0, 0)),
            pl.BlockSpec((1, D), lambda i: (0, 0)),
        ],
        out_specs=[
            pl.BlockSpec((EBLK, D), lambda i: (i, 0)),
            pl.BlockSpec((1, D), lambda i: (0, 0)),
            pl.BlockSpec((D, D), lambda i: (0, 0)),
        ],
        out_shape=[
            jax.ShapeDtypeStruct((E_PAD, D), jnp.bfloat16),
            jax.ShapeDtypeStruct((1, D), jnp.float32),
            jax.ShapeDtypeStruct((D, D), jnp.float32),
        ],
    )(GP, GQ, efp, Wc, bl1r, Wl2, bl2r)


def _fold_kernel(msum_ref, c_ref, wl3_ref, bl3_ref, ge_ref, be2_ref,
                 w3s_ref, b3s_ref):
    wl3 = wl3_ref[...]
    m = msum_ref[...] / E
    bl3 = bl3_ref[...]
    wm = lax.dot_general(wl3, m, (((1,), (0,)), ((), ())),
                         preferred_element_type=jnp.float32)
    mu_e = wm + bl3
    t = lax.dot_general(wl3, c_ref[...] / E, (((1,), (0,)), ((), ())),
                        preferred_element_type=jnp.float32)
    ex2 = jnp.sum(t * wl3, axis=1, keepdims=True) + 2.0 * bl3 * wm + bl3 * bl3
    var = ex2 - mu_e * mu_e
    s = ge_ref[...] * lax.rsqrt(var + 1e-5)
    w3s_ref[...] = s * wl3
    b3s_ref[...] = s * (bl3 - mu_e) + be2_ref[...]


def _tc_fold(msum_col, C, Wl3, bl3c, gec, be2c):
    return pl.pallas_call(
        _fold_kernel,
        out_shape=[
            jax.ShapeDtypeStruct((D, D), jnp.float32),
            jax.ShapeDtypeStruct((D, 1), jnp.float32),
        ],
    )(msum_col, C, Wl3, bl3c, gec, be2c)


def _edge3_kernel(f2_ref, w3s_ref, b3s_ref, o_ref):
    o_ref[...] = jnp.maximum(
        _dgT(f2_ref[...].astype(jnp.float32), w3s_ref[...]) + b3s_ref[...],
        0.0)


def _tc_edge3(f2, W3s, b3sr):
    return pl.pallas_call(
        _edge3_kernel,
        grid=(E_PAD // EBLK,),
        in_specs=[
            pl.BlockSpec((EBLK, D), lambda i: (i, 0)),
            pl.BlockSpec((D, D), lambda i: (0, 0)),
            pl.BlockSpec((1, D), lambda i: (0, 0)),
        ],
        out_specs=pl.BlockSpec((EBLK, D), lambda i: (i, 0)),
        out_shape=jax.ShapeDtypeStruct((E_PAD, D), jnp.float32),
    )(f2, W3s, b3sr)


def _pack16(x):
    return lax.bitcast_convert_type(
        x.reshape(x.shape[0], x.shape[1] // 2, 2), jnp.int32)


def _unpack16(x):
    y = lax.bitcast_convert_type(x, jnp.bfloat16)
    return y.reshape(x.shape[0], x.shape[1] * 2)


def kernel(em, edge_index, edge_features, W1, b1, W2, b2, We, be,
           Wl1, bl1, Wl2, bl2, Wl3, bl3, gx, bx, ge, be2):
    src = edge_index[0].astype(jnp.int32)
    dst = edge_index[1].astype(jnp.int32)
    pad = E_PAD - E
    zpad = jnp.zeros((pad,), jnp.int32)
    src_g = jnp.concatenate([src, zpad])
    dst_g = jnp.concatenate([dst, zpad])
    dst_s = jnp.concatenate([dst, jnp.full((pad,), N, jnp.int32)])
    efp = jnp.concatenate(
        [edge_features, jnp.zeros((pad, ED), jnp.float32)], axis=0)
    zeros_acc = jnp.zeros((ACC_ROWS, D), jnp.float32)

    Wa = Wl1[:, :D]
    Wb = Wl1[:, D:2 * D]
    Wc = Wl1[:, 2 * D:]

    em_p = jnp.concatenate(
        [em.astype(jnp.bfloat16), jnp.zeros((NPAD - N, D), jnp.bfloat16)],
        axis=0)
    G = _unpack16(_sc_gather(_pack16(em_p), src_g))
    msg = _tc_msg(G, efp, We, be.reshape(1, D))
    parts = _sc_scatter_add(msg, dst_s, zeros_acc)

    x_em, P, Q = _tc_node(em, parts, W1, b1.reshape(1, D), W2,
                          b2.reshape(1, D), gx.reshape(1, D),
                          bx.reshape(1, D), Wa, Wb)

    GP = _unpack16(_sc_gather(_pack16(P), src_g))
    GQ = _unpack16(_sc_gather(_pack16(Q), dst_g))

    f2, msum, C = _tc_edge12(GP, GQ, efp, Wc, bl1.reshape(1, D), Wl2,
                             bl2.reshape(1, D))

    W3s, b3s = _tc_fold(msum.reshape(D, 1), C, Wl3, bl3.reshape(D, 1),
                        ge.reshape(D, 1), be2.reshape(D, 1))
    edge_out = _tc_edge3(f2, W3s, b3s.reshape(1, D))

    return (x_em, edge_out[:E])


# speedup vs baseline: 2.3881x; 2.3881x over previous
import functools

import jax
import jax.numpy as jnp
from jax import lax
from jax.experimental import pallas as pl
from jax.experimental.pallas import tpu as pltpu
from jax.experimental.pallas import tpu_sc as plsc

N = 10000
E = 320000
D = 128
ED = 16

NC = 2
NS = 16
NW = NC * NS
SC_BLK = 128
E_PAD = 327680
PER_W = E_PAD // NW
NBLK = PER_W // SC_BLK
ROWS_PER_SUB = 632
ACC_ROWS = NS * ROWS_PER_SUB

EBLK = 1280
N_REAL_BLOCKS = E // EBLK
NPAD = 10240

def _mesh():
    return plsc.VectorSubcoreMesh(core_axis_name="c", subcore_axis_name="s",
                                  num_cores=NC)


def _dgT(x, w):
    return lax.dot_general(x, w, (((1,), (1,)), ((), ())),
                           preferred_element_type=jnp.float32)


NBG = 2


def _sc_gather(table, idx):
    T, W = table.shape
    rows_per_sub = T // NS
    assert T % (16 * NS) == 0

    dt = table.dtype

    @functools.partial(
        pl.kernel,
        out_type=jax.ShapeDtypeStruct((E_PAD, W), dt),
        mesh=_mesh(),
        scratch_types=[
            pltpu.VMEM((NBG, SC_BLK), jnp.int32),
            pltpu.VMEM((NBG, SC_BLK, W), dt),
            pltpu.VMEM_SHARED((T, W), dt),
        ] + [pltpu.SemaphoreType.DMA] * (3 * NBG),
    )
    def k(table_hbm, idx_hbm, out_hbm, idxs, bufs, tab_sh, *sems):
        isem = sems[:NBG]
        gsem = sems[NBG:2 * NBG]
        ssem = sems[2 * NBG:]
        cid = lax.axis_index("c")
        sid = lax.axis_index("s")
        wid = sid * NC + cid
        base = pl.multiple_of(wid * PER_W, SC_BLK)

        r0 = pl.multiple_of(sid * rows_per_sub, 16)
        pltpu.sync_copy(table_hbm.at[pl.ds(r0, rows_per_sub)],
                        tab_sh.at[pl.ds(r0, rows_per_sub)])

        def start_idx(blk, b):
            off = pl.multiple_of(base + blk * SC_BLK, SC_BLK)
            pltpu.make_async_copy(idx_hbm.at[pl.ds(off, SC_BLK)],
                                  idxs.at[b], isem[b]).start()

        for b in range(NBG):
            start_idx(b, b)

        plsc.subcore_barrier()

        @pl.loop(0, NBLK, step=NBG)
        def _(i):
            for b in range(NBG):
                blk = i + b
                off = pl.multiple_of(base + blk * SC_BLK, SC_BLK)
                pltpu.make_async_copy(idx_hbm.at[pl.ds(off, SC_BLK)],
                                      idxs.at[b], isem[b]).wait()
                pltpu.make_async_copy(tab_sh.at[idxs.at[b]], bufs.at[b],
                                      gsem[b]).start()
            for b in range(NBG):
                pltpu.make_async_copy(tab_sh.at[idxs.at[b]], bufs.at[b],
                                      gsem[b]).wait()
                off = pl.multiple_of(base + (i + b) * SC_BLK, SC_BLK)
                pltpu.make_async_copy(bufs.at[b],
                                      out_hbm.at[pl.ds(off, SC_BLK)],
                                      ssem[b]).start()
            for b in range(NBG):
                blk = i + b
                off = pl.multiple_of(base + blk * SC_BLK, SC_BLK)
                pltpu.make_async_copy(bufs.at[b],
                                      out_hbm.at[pl.ds(off, SC_BLK)],
                                      ssem[b]).wait()
                nxt = blk + NBG
                nxt = jnp.where(nxt >= NBLK, nxt - NBLK, nxt)
                start_idx(nxt, b)

        for b in range(NBG):
            pltpu.make_async_copy(idx_hbm.at[pl.ds(base, SC_BLK)],
                                  idxs.at[b], isem[b]).wait()

    return k(table, idx)


def _sc_scatter_add(msg, dst, zeros):
    NBS = 2

    @functools.partial(
        pl.kernel,
        out_type=jax.ShapeDtypeStruct((NC, ACC_ROWS, D), jnp.float32),
        mesh=_mesh(),
        scratch_types=[
            pltpu.VMEM((NBS, SC_BLK), jnp.int32),
            pltpu.VMEM((NBS, SC_BLK, D), jnp.float32),
            pltpu.VMEM_SHARED((ACC_ROWS, D), jnp.float32),
        ] + [pltpu.SemaphoreType.DMA] * (2 * NBS),
    )
    def k(msg_hbm, dst_hbm, zero_hbm, out_hbm, idxs, bufs, acc_sh, *sems):
        isem, msem = sems[:NBS], sems[NBS:]
        cid = lax.axis_index("c")
        sid = lax.axis_index("s")
        zoff = pl.multiple_of(sid * ROWS_PER_SUB, 8)
        pltpu.sync_copy(zero_hbm.at[pl.ds(zoff, ROWS_PER_SUB)],
                        acc_sh.at[pl.ds(zoff, ROWS_PER_SUB)])

        base = pl.multiple_of(cid * (E_PAD // NC) + sid * PER_W, SC_BLK)

        def start_loads(blk, b):
            off = pl.multiple_of(base + blk * SC_BLK, SC_BLK)
            pltpu.make_async_copy(dst_hbm.at[pl.ds(off, SC_BLK)],
                                  idxs.at[b], isem[b]).start()
            pltpu.make_async_copy(msg_hbm.at[pl.ds(off, SC_BLK)],
                                  bufs.at[b], msem[b]).start()

        for b in range(NBS):
            start_loads(b, b)

        plsc.subcore_barrier()

        @pl.loop(0, NBLK, step=NBS)
        def _(i):
            for b in range(NBS):
                blk = i + b
                off = pl.multiple_of(base + blk * SC_BLK, SC_BLK)
                pltpu.make_async_copy(dst_hbm.at[pl.ds(off, SC_BLK)],
                                      idxs.at[b], isem[b]).wait()
                pltpu.make_async_copy(msg_hbm.at[pl.ds(off, SC_BLK)],
                                      bufs.at[b], msem[b]).wait()
                pltpu.sync_copy(bufs.at[b], acc_sh.at[idxs.at[b]], add=True)
                nxt = blk + NBS
                nxt = jnp.where(nxt >= NBLK, nxt - NBLK, nxt)
                start_loads(nxt, b)

        for b in range(NBS):
            off = pl.multiple_of(base + b * SC_BLK, SC_BLK)
            pltpu.make_async_copy(dst_hbm.at[pl.ds(off, SC_BLK)],
                                  idxs.at[b], isem[b]).wait()
            pltpu.make_async_copy(msg_hbm.at[pl.ds(off, SC_BLK)],
                                  bufs.at[b], msem[b]).wait()

        plsc.subcore_barrier()
        pltpu.sync_copy(acc_sh.at[pl.ds(zoff, ROWS_PER_SUB)],
                        out_hbm.at[cid].at[pl.ds(zoff, ROWS_PER_SUB)])

    return k(msg, dst, zeros)


def _msg_kernel(g_ref, ef_ref, we_ref, be_ref, o_ref):
    g = g_ref[...].astype(jnp.float32)
    o_ref[...] = jnp.maximum(
        g + _dgT(ef_ref[...], we_ref[...]) + be_ref[...], 0.0)


def _tc_msg(G, efp, We, be2d):
    return pl.pallas_call(
        _msg_kernel,
        grid=(E_PAD // EBLK,),
        in_specs=[
            pl.BlockSpec((EBLK, D), lambda i: (i, 0)),
            pl.BlockSpec((EBLK, ED), lambda i: (i, 0)),
            pl.BlockSpec((D, ED), lambda i: (0, 0)),
            pl.BlockSpec((1, D), lambda i: (0, 0)),
        ],
        out_specs=pl.BlockSpec((EBLK, D), lambda i: (i, 0)),
        out_shape=jax.ShapeDtypeStruct((E_PAD, D), jnp.float32),
    )(G, efp, We, be2d)


def _node_kernel(em_ref, parts_ref, w1_ref, b1_ref, w2_ref, b2_ref,
                 gx_ref, bx_ref, wa_ref, wb_ref, xem_ref, p_ref, q_ref):
    h = em_ref[...] + parts_ref[0, :N, :] + parts_ref[1, :N, :]
    h = jnp.maximum(_dgT(h, w1_ref[...]) + b1_ref[...], 0.0)
    h = _dgT(h, w2_ref[...]) + b2_ref[...]
    mu = jnp.mean(h, axis=0, keepdims=True)
    var = jnp.mean((h - mu) ** 2, axis=0, keepdims=True)
    xem = (h - mu) * lax.rsqrt(var + 1e-5) * gx_ref[...] + bx_ref[...]
    xem = jnp.maximum(xem, 0.0)
    xem_ref[...] = xem
    p_ref[:N, :] = _dgT(xem, wa_ref[...])
    q_ref[:N, :] = _dgT(xem, wb_ref[...])


def _tc_node(em, parts, W1, b1r, W2, b2r, gxr, bxr, Wa, Wb):
    return pl.pallas_call(
        _node_kernel,
        out_shape=[
            jax.ShapeDtypeStruct((N, D), jnp.float32),
            jax.ShapeDtypeStruct((NPAD, D), jnp.float32),
            jax.ShapeDtypeStruct((NPAD, D), jnp.float32),
        ],
    )(em, parts, W1, b1r, W2, b2r, gxr, bxr, Wa, Wb)


def _edge12_kernel(gp_ref, gq_ref, ef_ref, wc_ref, bl1_ref, wl2_ref, bl2_ref,
                   f2_ref, msum_ref, c_ref):
    pid = pl.program_id(0)

    @pl.when(pid == 0)
    def _():
        msum_ref[...] = jnp.zeros_like(msum_ref)
        c_ref[...] = jnp.zeros_like(c_ref)

    gpq = gp_ref[...].astype(jnp.float32) + gq_ref[...].astype(jnp.float32)
    f1 = jnp.maximum(
        gpq + _dgT(ef_ref[...], wc_ref[...]) + bl1_ref[...], 0.0)
    f2 = jnp.maximum(_dgT(f1, wl2_ref[...]) + bl2_ref[...], 0.0)
    f2_ref[...] = f2.astype(jnp.bfloat16)

    @pl.when(pid < N_REAL_BLOCKS)
    def _():
        msum_ref[...] += jnp.sum(f2, axis=0, keepdims=True)
        c_ref[...] += lax.dot_general(f2, f2, (((0,), (0,)), ((), ())),
                                      preferred_element_type=jnp.float32)


def _tc_edge12(GP, GQ, efp, Wc, bl1r, Wl2, bl2r):
    return pl.pallas_call(
        _edge12_kernel,
        grid=(E_PAD // EBLK,),
        in_specs=[
            pl.BlockSpec((EBLK, D), lambda i: (i, 0)),
            pl.BlockSpec((EBLK, D), lambda i: (i, 0)),
            pl.BlockSpec((EBLK, ED), lambda i: (i, 0)),
            pl.BlockSpec((D, ED), lambda i: (0, 0)),
            pl.BlockSpec((1, D), lambda i: (0, 0)),
            pl.BlockSpec((D, D), lambda i: (0, 0)),
            pl.BlockSpec((1, D), lambda i: (0, 0)),
        ],
        out_specs=[
            pl.BlockSpec((EBLK, D), lambda i: (i, 0)),
            pl.BlockSpec((1, D), lambda i: (0, 0)),
            pl.BlockSpec((D, D), lambda i: (0, 0)),
        ],
        out_shape=[
            jax.ShapeDtypeStruct((E_PAD, D), jnp.bfloat16),
            jax.ShapeDtypeStruct((1, D), jnp.float32),
            jax.ShapeDtypeStruct((D, D), jnp.float32),
        ],
    )(GP, GQ, efp, Wc, bl1r, Wl2, bl2r)


def _fold_kernel(msum_ref, c_ref, wl3_ref, bl3_ref, ge_ref, be2_ref,
                 w3s_ref, b3s_ref):
    wl3 = wl3_ref[...]
    m = msum_ref[...] / E
    bl3 = bl3_ref[...]
    wm = lax.dot_general(wl3, m, (((1,), (0,)), ((), ())),
                         preferred_element_type=jnp.float32)
    mu_e = wm + bl3
    t = lax.dot_general(wl3, c_ref[...] / E, (((1,), (0,)), ((), ())),
                        preferred_element_type=jnp.float32)
    ex2 = jnp.sum(t * wl3, axis=1, keepdims=True) + 2.0 * bl3 * wm + bl3 * bl3
    var = ex2 - mu_e * mu_e
    s = ge_ref[...] * lax.rsqrt(var + 1e-5)
    w3s_ref[...] = s * wl3
    b3s_ref[...] = s * (bl3 - mu_e) + be2_ref[...]


def _tc_fold(msum_col, C, Wl3, bl3c, gec, be2c):
    return pl.pallas_call(
        _fold_kernel,
        out_shape=[
            jax.ShapeDtypeStruct((D, D), jnp.float32),
            jax.ShapeDtypeStruct((D, 1), jnp.float32),
        ],
    )(msum_col, C, Wl3, bl3c, gec, be2c)


def _edge3_kernel(f2_ref, w3s_ref, b3s_ref, o_ref):
    o_ref[...] = jnp.maximum(
        _dgT(f2_ref[...].astype(jnp.float32), w3s_ref[...]) + b3s_ref[...],
        0.0)


def _tc_edge3(f2, W3s, b3sr):
    return pl.pallas_call(
        _edge3_kernel,
        grid=(E_PAD // EBLK,),
        in_specs=[
            pl.BlockSpec((EBLK, D), lambda i: (i, 0)),
            pl.BlockSpec((D, D), lambda i: (0, 0)),
            pl.BlockSpec((1, D), lambda i: (0, 0)),
        ],
        out_specs=pl.BlockSpec((EBLK, D), lambda i: (i, 0)),
        out_shape=jax.ShapeDtypeStruct((E_PAD, D), jnp.float32),
    )(f2, W3s, b3sr)


def kernel(em, edge_index, edge_features, W1, b1, W2, b2, We, be,
           Wl1, bl1, Wl2, bl2, Wl3, bl3, gx, bx, ge, be2):
    src = edge_index[0].astype(jnp.int32)
    dst = edge_index[1].astype(jnp.int32)
    pad = E_PAD - E
    zpad = jnp.zeros((pad,), jnp.int32)
    src_g = jnp.concatenate([src, zpad])
    dst_g = jnp.concatenate([dst, zpad])
    dst_s = jnp.concatenate([dst, jnp.full((pad,), N, jnp.int32)])
    efp = jnp.concatenate(
        [edge_features, jnp.zeros((pad, ED), jnp.float32)], axis=0)
    zeros_acc = jnp.zeros((ACC_ROWS, D), jnp.float32)

    Wa = Wl1[:, :D]
    Wb = Wl1[:, D:2 * D]
    Wc = Wl1[:, 2 * D:]

    em_p = jnp.concatenate([em, jnp.zeros((NPAD - N, D), jnp.float32)],
                           axis=0)
    G = _sc_gather(em_p, src_g)
    msg = _tc_msg(G, efp, We, be.reshape(1, D))
    parts = _sc_scatter_add(msg, dst_s, zeros_acc)

    x_em, P, Q = _tc_node(em, parts, W1, b1.reshape(1, D), W2,
                          b2.reshape(1, D), gx.reshape(1, D),
                          bx.reshape(1, D), Wa, Wb)

    GP = _sc_gather(P, src_g)
    GQ = _sc_gather(Q, dst_g)

    f2, msum, C = _tc_edge12(GP, GQ, efp, Wc, bl1.reshape(1, D), Wl2,
                             bl2.reshape(1, D))

    W3s, b3s = _tc_fold(msum.reshape(D, 1), C, Wl3, bl3.reshape(D, 1),
                        ge.reshape(D, 1), be2.reshape(D, 1))
    edge_out = _tc_edge3(f2, W3s, b3s.reshape(1, D))

    return (x_em, edge_out[:E])


# speedup vs baseline: 2.4251x; 1.0155x over previous
import functools

import jax
import jax.numpy as jnp
from jax import lax
from jax.experimental import pallas as pl
from jax.experimental.pallas import tpu as pltpu
from jax.experimental.pallas import tpu_sc as plsc

N = 10000
E = 320000
D = 128
ED = 16

NC = 2
NS = 16
NW = NC * NS
SC_BLK = 128
E_PAD = 327680
PER_W = E_PAD // NW
NBLK = PER_W // SC_BLK
ROWS_PER_SUB = 632
ACC_ROWS = NS * ROWS_PER_SUB

EBLK = 1280
N_REAL_BLOCKS = E // EBLK
NPAD = 10240

def _mesh():
    return plsc.VectorSubcoreMesh(core_axis_name="c", subcore_axis_name="s",
                                  num_cores=NC)


def _dgT(x, w):
    return lax.dot_general(x, w, (((1,), (1,)), ((), ())),
                           preferred_element_type=jnp.float32)


NBG = 2


def _sc_gather(table, idx):
    T, W = table.shape
    rows_per_sub = T // NS
    assert T % (16 * NS) == 0
    e_len = idx.shape[0]
    per_w = e_len // NW
    nblk = per_w // SC_BLK
    assert e_len == per_w * NW and per_w % SC_BLK == 0

    dt = table.dtype

    @functools.partial(
        pl.kernel,
        out_type=jax.ShapeDtypeStruct((e_len, W), dt),
        mesh=_mesh(),
        scratch_types=[
            pltpu.VMEM((NBG, SC_BLK), jnp.int32),
            pltpu.VMEM((NBG, SC_BLK, W), dt),
            pltpu.VMEM_SHARED((T, W), dt),
        ] + [pltpu.SemaphoreType.DMA] * (3 * NBG),
    )
    def k(table_hbm, idx_hbm, out_hbm, idxs, bufs, tab_sh, *sems):
        isem = sems[:NBG]
        gsem = sems[NBG:2 * NBG]
        ssem = sems[2 * NBG:]
        cid = lax.axis_index("c")
        sid = lax.axis_index("s")
        wid = sid * NC + cid
        base = pl.multiple_of(wid * per_w, SC_BLK)

        r0 = pl.multiple_of(sid * rows_per_sub, 16)
        pltpu.sync_copy(table_hbm.at[pl.ds(r0, rows_per_sub)],
                        tab_sh.at[pl.ds(r0, rows_per_sub)])

        def start_idx(blk, b):
            off = pl.multiple_of(base + blk * SC_BLK, SC_BLK)
            pltpu.make_async_copy(idx_hbm.at[pl.ds(off, SC_BLK)],
                                  idxs.at[b], isem[b]).start()

        def store_op(blk, b):
            off = pl.multiple_of(base + blk * SC_BLK, SC_BLK)
            return pltpu.make_async_copy(
                bufs.at[b], out_hbm.at[pl.ds(off, SC_BLK)], ssem[b])

        for b in range(NBG):
            start_idx(b, b)
            store_op(b, b).start()

        plsc.subcore_barrier()

        @pl.loop(0, nblk, step=NBG)
        def _(i):
            for b in range(NBG):
                blk = i + b
                store_op(blk, b).wait()
                off = pl.multiple_of(base + blk * SC_BLK, SC_BLK)
                pltpu.make_async_copy(idx_hbm.at[pl.ds(off, SC_BLK)],
                                      idxs.at[b], isem[b]).wait()
                pltpu.make_async_copy(tab_sh.at[idxs.at[b]], bufs.at[b],
                                      gsem[b]).start()
            for b in range(NBG):
                blk = i + b
                pltpu.make_async_copy(tab_sh.at[idxs.at[b]], bufs.at[b],
                                      gsem[b]).wait()
                store_op(blk, b).start()
                nxt = blk + NBG
                nxt = jnp.where(nxt >= nblk, nxt - nblk, nxt)
                start_idx(nxt, b)

        for b in range(NBG):
            store_op(b, b).wait()
            pltpu.make_async_copy(idx_hbm.at[pl.ds(base, SC_BLK)],
                                  idxs.at[b], isem[b]).wait()

    return k(table, idx)


def _sc_scatter_add(msg, dst, zeros):
    e_len = msg.shape[0]
    per_w = e_len // NW
    nblk = per_w // SC_BLK
    assert e_len == per_w * NW and per_w % SC_BLK == 0

    NBS = 2

    @functools.partial(
        pl.kernel,
        out_type=jax.ShapeDtypeStruct((NC, ACC_ROWS, D), jnp.float32),
        mesh=_mesh(),
        scratch_types=[
            pltpu.VMEM((NBS, SC_BLK), jnp.int32),
            pltpu.VMEM((NBS, SC_BLK, D), jnp.float32),
            pltpu.VMEM_SHARED((ACC_ROWS, D), jnp.float32),
        ] + [pltpu.SemaphoreType.DMA] * (2 * NBS),
    )
    def k(msg_hbm, dst_hbm, zero_hbm, out_hbm, idxs, bufs, acc_sh, *sems):
        isem, msem = sems[:NBS], sems[NBS:]
        cid = lax.axis_index("c")
        sid = lax.axis_index("s")
        zoff = pl.multiple_of(sid * ROWS_PER_SUB, 8)
        pltpu.sync_copy(zero_hbm.at[pl.ds(zoff, ROWS_PER_SUB)],
                        acc_sh.at[pl.ds(zoff, ROWS_PER_SUB)])

        base = pl.multiple_of(cid * (e_len // NC) + sid * per_w, SC_BLK)

        def start_loads(blk, b):
            off = pl.multiple_of(base + blk * SC_BLK, SC_BLK)
            pltpu.make_async_copy(dst_hbm.at[pl.ds(off, SC_BLK)],
                                  idxs.at[b], isem[b]).start()
            pltpu.make_async_copy(msg_hbm.at[pl.ds(off, SC_BLK)],
                                  bufs.at[b], msem[b]).start()

        for b in range(NBS):
            start_loads(b, b)

        plsc.subcore_barrier()

        @pl.loop(0, nblk, step=NBS)
        def _(i):
            for b in range(NBS):
                blk = i + b
                off = pl.multiple_of(base + blk * SC_BLK, SC_BLK)
                pltpu.make_async_copy(dst_hbm.at[pl.ds(off, SC_BLK)],
                                      idxs.at[b], isem[b]).wait()
                pltpu.make_async_copy(msg_hbm.at[pl.ds(off, SC_BLK)],
                                      bufs.at[b], msem[b]).wait()
                pltpu.sync_copy(bufs.at[b], acc_sh.at[idxs.at[b]], add=True)
                nxt = blk + NBS
                nxt = jnp.where(nxt >= nblk, nxt - nblk, nxt)
                start_loads(nxt, b)

        for b in range(NBS):
            off = pl.multiple_of(base + b * SC_BLK, SC_BLK)
            pltpu.make_async_copy(dst_hbm.at[pl.ds(off, SC_BLK)],
                                  idxs.at[b], isem[b]).wait()
            pltpu.make_async_copy(msg_hbm.at[pl.ds(off, SC_BLK)],
                                  bufs.at[b], msem[b]).wait()

        plsc.subcore_barrier()
        pltpu.sync_copy(acc_sh.at[pl.ds(zoff, ROWS_PER_SUB)],
                        out_hbm.at[cid].at[pl.ds(zoff, ROWS_PER_SUB)])

    return k(msg, dst, zeros)


def _msg_kernel(g_ref, ef_ref, we_ref, be_ref, o_ref):
    g = g_ref[...].astype(jnp.float32)
    o_ref[...] = jnp.maximum(
        g + _dgT(ef_ref[...], we_ref[...]) + be_ref[...], 0.0)


def _tc_msg(G, efp, We, be2d):
    e_len = G.shape[0]
    return pl.pallas_call(
        _msg_kernel,
        grid=(e_len // EBLK,),
        in_specs=[
            pl.BlockSpec((EBLK, D), lambda i: (i, 0)),
            pl.BlockSpec((EBLK, ED), lambda i: (i, 0)),
            pl.BlockSpec((D, ED), lambda i: (0, 0)),
            pl.BlockSpec((1, D), lambda i: (0, 0)),
        ],
        out_specs=pl.BlockSpec((EBLK, D), lambda i: (i, 0)),
        out_shape=jax.ShapeDtypeStruct((e_len, D), jnp.float32),
    )(G, efp, We, be2d)


def _node_kernel(em_ref, pa_ref, pb_ref, w1_ref, b1_ref, w2_ref, b2_ref,
                 gx_ref, bx_ref, wa_ref, wb_ref, xem_ref, p_ref, q_ref):
    h = (em_ref[...] + pa_ref[0, :N, :] + pa_ref[1, :N, :]
         + pb_ref[0, :N, :] + pb_ref[1, :N, :])
    h = jnp.maximum(_dgT(h, w1_ref[...]) + b1_ref[...], 0.0)
    h = _dgT(h, w2_ref[...]) + b2_ref[...]
    mu = jnp.mean(h, axis=0, keepdims=True)
    var = jnp.mean((h - mu) ** 2, axis=0, keepdims=True)
    xem = (h - mu) * lax.rsqrt(var + 1e-5) * gx_ref[...] + bx_ref[...]
    xem = jnp.maximum(xem, 0.0)
    xem_ref[...] = xem
    p_ref[:N, :] = _dgT(xem, wa_ref[...])
    q_ref[:N, :] = _dgT(xem, wb_ref[...])


def _tc_node(em, parts_a, parts_b, W1, b1r, W2, b2r, gxr, bxr, Wa, Wb):
    return pl.pallas_call(
        _node_kernel,
        out_shape=[
            jax.ShapeDtypeStruct((N, D), jnp.float32),
            jax.ShapeDtypeStruct((NPAD, D), jnp.float32),
            jax.ShapeDtypeStruct((NPAD, D), jnp.float32),
        ],
    )(em, parts_a, parts_b, W1, b1r, W2, b2r, gxr, bxr, Wa, Wb)


def _edge12_kernel(gp_ref, gq_ref, ef_ref, wc_ref, bl1_ref, wl2_ref, bl2_ref,
                   f2_ref, msum_ref, c_ref):
    pid = pl.program_id(0)

    @pl.when(pid == 0)
    def _():
        msum_ref[...] = jnp.zeros_like(msum_ref)
        c_ref[...] = jnp.zeros_like(c_ref)

    gpq = gp_ref[...].astype(jnp.float32) + gq_ref[...].astype(jnp.float32)
    f1 = jnp.maximum(
        gpq + _dgT(ef_ref[...], wc_ref[...]) + bl1_ref[...], 0.0)
    f2 = jnp.maximum(_dgT(f1, wl2_ref[...]) + bl2_ref[...], 0.0)
    f2_ref[...] = f2.astype(jnp.bfloat16)

    @pl.when(pid < N_REAL_BLOCKS)
    def _():
        msum_ref[...] += jnp.sum(f2, axis=0, keepdims=True)
        c_ref[...] += lax.dot_general(f2, f2, (((0,), (0,)), ((), ())),
                                      preferred_element_type=jnp.float32)


def _tc_edge12(GP, GQ, efp, Wc, bl1r, Wl2, bl2r):
    return pl.pallas_call(
        _edge12_kernel,
        grid=(E_PAD // EBLK,),
        in_specs=[
            pl.BlockSpec((EBLK, D), lambda i: (i, 0)),
            pl.BlockSpec((EBLK, D), lambda i: (i, 0)),
            pl.BlockSpec((EBLK, ED), lambda i: (i, 0)),
            pl.BlockSpec((D, ED), lambda i: (0, 0)),
            pl.BlockSpec((1, D), lambda i: (0, 0)),
            pl.BlockSpec((D, D), lambda i: (0, 0)),
            pl.BlockSpec((1, D), lambda i: (0, 0)),
        ],
        out_specs=[
            pl.BlockSpec((EBLK, D), lambda i: (i, 0)),
            pl.BlockSpec((1, D), lambda i: (0, 0)),
            pl.BlockSpec((D, D), lambda i: (0, 0)),
        ],
        out_shape=[
            jax.ShapeDtypeStruct((E_PAD, D), jnp.bfloat16),
            jax.ShapeDtypeStruct((1, D), jnp.float32),
            jax.ShapeDtypeStruct((D, D), jnp.float32),
        ],
    )(GP, GQ, efp, Wc, bl1r, Wl2, bl2r)


def _fold_kernel(msum_ref, c_ref, wl3_ref, bl3_ref, ge_ref, be2_ref,
                 w3s_ref, b3s_ref):
    wl3 = wl3_ref[...]
    m = msum_ref[...] / E
    bl3 = bl3_ref[...]
    wm = lax.dot_general(wl3, m, (((1,), (0,)), ((), ())),
                         preferred_element_type=jnp.float32)
    mu_e = wm + bl3
    t = lax.dot_general(wl3, c_ref[...] / E, (((1,), (0,)), ((), ())),
                        preferred_element_type=jnp.float32)
    ex2 = jnp.sum(t * wl3, axis=1, keepdims=True) + 2.0 * bl3 * wm + bl3 * bl3
    var = ex2 - mu_e * mu_e
    s = ge_ref[...] * lax.rsqrt(var + 1e-5)
    w3s_ref[...] = s * wl3
    b3s_ref[...] = s * (bl3 - mu_e) + be2_ref[...]


def _tc_fold(msum_col, C, Wl3, bl3c, gec, be2c):
    return pl.pallas_call(
        _fold_kernel,
        out_shape=[
            jax.ShapeDtypeStruct((D, D), jnp.float32),
            jax.ShapeDtypeStruct((D, 1), jnp.float32),
        ],
    )(msum_col, C, Wl3, bl3c, gec, be2c)


def _edge3_kernel(f2_ref, w3s_ref, b3s_ref, o_ref):
    o_ref[...] = jnp.maximum(
        _dgT(f2_ref[...].astype(jnp.float32), w3s_ref[...]) + b3s_ref[...],
        0.0)


def _tc_edge3(f2, W3s, b3sr):
    return pl.pallas_call(
        _edge3_kernel,
        grid=(E_PAD // EBLK,),
        in_specs=[
            pl.BlockSpec((EBLK, D), lambda i: (i, 0)),
            pl.BlockSpec((D, D), lambda i: (0, 0)),
            pl.BlockSpec((1, D), lambda i: (0, 0)),
        ],
        out_specs=pl.BlockSpec((EBLK, D), lambda i: (i, 0)),
        out_shape=jax.ShapeDtypeStruct((E_PAD, D), jnp.float32),
    )(f2, W3s, b3sr)


def kernel(em, edge_index, edge_features, W1, b1, W2, b2, We, be,
           Wl1, bl1, Wl2, bl2, Wl3, bl3, gx, bx, ge, be2):
    src = edge_index[0].astype(jnp.int32)
    dst = edge_index[1].astype(jnp.int32)
    pad = E_PAD - E
    zpad = jnp.zeros((pad,), jnp.int32)
    src_g = jnp.concatenate([src, zpad])
    dst_g = jnp.concatenate([dst, zpad])
    dst_s = jnp.concatenate([dst, jnp.full((pad,), N, jnp.int32)])
    efp = jnp.concatenate(
        [edge_features, jnp.zeros((pad, ED), jnp.float32)], axis=0)
    zeros_acc = jnp.zeros((ACC_ROWS, D), jnp.float32)

    Wa = Wl1[:, :D]
    Wb = Wl1[:, D:2 * D]
    Wc = Wl1[:, 2 * D:]

    em_p = jnp.concatenate([em, jnp.zeros((NPAD - N, D), jnp.float32)],
                           axis=0)
    H1 = E_PAD // 2
    be2d = be.reshape(1, D)
    G1 = _sc_gather(em_p, src_g[:H1])
    msg1 = _tc_msg(G1, efp[:H1], We, be2d)
    G2 = _sc_gather(em_p, src_g[H1:])
    parts1 = _sc_scatter_add(msg1, dst_s[:H1], zeros_acc)
    msg2 = _tc_msg(G2, efp[H1:], We, be2d)
    parts2 = _sc_scatter_add(msg2, dst_s[H1:], zeros_acc)

    x_em, P, Q = _tc_node(em, parts1, parts2, W1, b1.reshape(1, D), W2,
                          b2.reshape(1, D), gx.reshape(1, D),
                          bx.reshape(1, D), Wa, Wb)

    GP = _sc_gather(P, src_g)
    GQ = _sc_gather(Q, dst_g)

    f2, msum, C = _tc_edge12(GP, GQ, efp, Wc, bl1.reshape(1, D), Wl2,
                             bl2.reshape(1, D))

    W3s, b3s = _tc_fold(msum.reshape(D, 1), C, Wl3, bl3.reshape(D, 1),
                        ge.reshape(D, 1), be2.reshape(D, 1))
    edge_out = _tc_edge3(f2, W3s, b3s.reshape(1, D))

    return (x_em, edge_out[:E])


# speedup vs baseline: 2.4603x; 1.0145x over previous
import functools

import jax
import jax.numpy as jnp
from jax import lax
from jax.experimental import pallas as pl
from jax.experimental.pallas import tpu as pltpu
from jax.experimental.pallas import tpu_sc as plsc

N = 10000
E = 320000
D = 128
ED = 16

NC = 2
NS = 16
NW = NC * NS
SC_BLK = 128
E_PAD = 327680
PER_W = E_PAD // NW
NBLK = PER_W // SC_BLK
ROWS_PER_SUB = 632
ACC_ROWS = NS * ROWS_PER_SUB

EBLK = 1280
N_REAL_BLOCKS = E // EBLK
NPAD = 10240

def _mesh():
    return plsc.VectorSubcoreMesh(core_axis_name="c", subcore_axis_name="s",
                                  num_cores=NC)


def _dgT(x, w):
    return lax.dot_general(x, w, (((1,), (1,)), ((), ())),
                           preferred_element_type=jnp.float32)


NBG = 2


def _sc_gather(table, idx):
    T, W = table.shape
    rows_per_sub = T // NS
    assert T % (16 * NS) == 0
    e_len = idx.shape[0]
    per_w = e_len // NW
    nblk = per_w // SC_BLK
    assert e_len == per_w * NW and per_w % SC_BLK == 0

    dt = table.dtype

    @functools.partial(
        pl.kernel,
        out_type=jax.ShapeDtypeStruct((e_len, W), dt),
        mesh=_mesh(),
        scratch_types=[
            pltpu.VMEM((NBG, SC_BLK), jnp.int32),
            pltpu.VMEM((NBG, SC_BLK, W), dt),
            pltpu.VMEM_SHARED((T, W), dt),
        ] + [pltpu.SemaphoreType.DMA] * (3 * NBG),
    )
    def k(table_hbm, idx_hbm, out_hbm, idxs, bufs, tab_sh, *sems):
        isem = sems[:NBG]
        gsem = sems[NBG:2 * NBG]
        ssem = sems[2 * NBG:]
        cid = lax.axis_index("c")
        sid = lax.axis_index("s")
        wid = sid * NC + cid
        base = pl.multiple_of(wid * per_w, SC_BLK)

        r0 = pl.multiple_of(sid * rows_per_sub, 16)
        pltpu.sync_copy(table_hbm.at[pl.ds(r0, rows_per_sub)],
                        tab_sh.at[pl.ds(r0, rows_per_sub)])

        def start_idx(blk, b):
            off = pl.multiple_of(base + blk * SC_BLK, SC_BLK)
            pltpu.make_async_copy(idx_hbm.at[pl.ds(off, SC_BLK)],
                                  idxs.at[b], isem[b]).start()

        def store_op(blk, b):
            off = pl.multiple_of(base + blk * SC_BLK, SC_BLK)
            return pltpu.make_async_copy(
                bufs.at[b], out_hbm.at[pl.ds(off, SC_BLK)], ssem[b])

        for b in range(NBG):
            start_idx(b, b)
            store_op(b, b).start()

        plsc.subcore_barrier()

        @pl.loop(0, nblk, step=NBG)
        def _(i):
            for b in range(NBG):
                blk = i + b
                store_op(blk, b).wait()
                off = pl.multiple_of(base + blk * SC_BLK, SC_BLK)
                pltpu.make_async_copy(idx_hbm.at[pl.ds(off, SC_BLK)],
                                      idxs.at[b], isem[b]).wait()
                pltpu.make_async_copy(tab_sh.at[idxs.at[b]], bufs.at[b],
                                      gsem[b]).start()
            for b in range(NBG):
                blk = i + b
                pltpu.make_async_copy(tab_sh.at[idxs.at[b]], bufs.at[b],
                                      gsem[b]).wait()
                store_op(blk, b).start()
                nxt = blk + NBG
                nxt = jnp.where(nxt >= nblk, nxt - nblk, nxt)
                start_idx(nxt, b)

        for b in range(NBG):
            store_op(b, b).wait()
            pltpu.make_async_copy(idx_hbm.at[pl.ds(base, SC_BLK)],
                                  idxs.at[b], isem[b]).wait()

    return k(table, idx)


def _sc_gather_pq(P, Q, src, dst):
    T = P.shape[0]
    rows_per_sub = T // NS
    e_len = src.shape[0]
    per_w = e_len // NS
    nblk = per_w // SC_BLK
    assert e_len == per_w * NS and per_w % SC_BLK == 0

    @functools.partial(
        pl.kernel,
        out_type=[jax.ShapeDtypeStruct((e_len, D), jnp.float32)] * 2,
        mesh=_mesh(),
        scratch_types=[
            pltpu.VMEM((NBG, SC_BLK), jnp.int32),
            pltpu.VMEM((NBG, SC_BLK, D), jnp.float32),
            pltpu.VMEM_SHARED((T, D), jnp.float32),
        ] + [pltpu.SemaphoreType.DMA] * (3 * NBG),
    )
    def k(p_hbm, q_hbm, src_hbm, dst_hbm, gp_hbm, gq_hbm,
          idxs, bufs, tab_sh, *sems):
        isem = sems[:NBG]
        gsem = sems[NBG:2 * NBG]
        ssem = sems[2 * NBG:]
        cid = lax.axis_index("c")
        sid = lax.axis_index("s")
        base = pl.multiple_of(sid * per_w, SC_BLK)
        r0 = pl.multiple_of(sid * rows_per_sub, 16)

        def pipeline(tab_hbm, idx_hbm, out_hbm):
            pltpu.sync_copy(tab_hbm.at[pl.ds(r0, rows_per_sub)],
                            tab_sh.at[pl.ds(r0, rows_per_sub)])

            def start_idx(blk, b):
                off = pl.multiple_of(base + blk * SC_BLK, SC_BLK)
                pltpu.make_async_copy(idx_hbm.at[pl.ds(off, SC_BLK)],
                                      idxs.at[b], isem[b]).start()

            def store_op(blk, b):
                off = pl.multiple_of(base + blk * SC_BLK, SC_BLK)
                return pltpu.make_async_copy(
                    bufs.at[b], out_hbm.at[pl.ds(off, SC_BLK)], ssem[b])

            for b in range(NBG):
                start_idx(b, b)
                store_op(b, b).start()

            plsc.subcore_barrier()

            @pl.loop(0, nblk, step=NBG)
            def _(i):
                for b in range(NBG):
                    blk = i + b
                    store_op(blk, b).wait()
                    off = pl.multiple_of(base + blk * SC_BLK, SC_BLK)
                    pltpu.make_async_copy(idx_hbm.at[pl.ds(off, SC_BLK)],
                                          idxs.at[b], isem[b]).wait()
                    pltpu.make_async_copy(tab_sh.at[idxs.at[b]], bufs.at[b],
                                          gsem[b]).start()
                for b in range(NBG):
                    blk = i + b
                    pltpu.make_async_copy(tab_sh.at[idxs.at[b]], bufs.at[b],
                                          gsem[b]).wait()
                    store_op(blk, b).start()
                    nxt = blk + NBG
                    nxt = jnp.where(nxt >= nblk, nxt - nblk, nxt)
                    start_idx(nxt, b)

            for b in range(NBG):
                store_op(b, b).wait()
                pltpu.make_async_copy(idx_hbm.at[pl.ds(base, SC_BLK)],
                                      idxs.at[b], isem[b]).wait()

        @pl.when(cid == 0)
        def _():
            pipeline(p_hbm, src_hbm, gp_hbm)

        @pl.when(cid == 1)
        def _():
            pipeline(q_hbm, dst_hbm, gq_hbm)

    return k(P, Q, src, dst)


def _sc_scatter_add(msg, dst, zeros):
    e_len = msg.shape[0]
    per_w = e_len // NW
    nblk = per_w // SC_BLK
    assert e_len == per_w * NW and per_w % SC_BLK == 0

    NBS = 2

    @functools.partial(
        pl.kernel,
        out_type=jax.ShapeDtypeStruct((NC, ACC_ROWS, D), jnp.float32),
        mesh=_mesh(),
        scratch_types=[
            pltpu.VMEM((NBS, SC_BLK), jnp.int32),
            pltpu.VMEM((NBS, SC_BLK, D), jnp.float32),
            pltpu.VMEM_SHARED((ACC_ROWS, D), jnp.float32),
        ] + [pltpu.SemaphoreType.DMA] * (2 * NBS),
    )
    def k(msg_hbm, dst_hbm, zero_hbm, out_hbm, idxs, bufs, acc_sh, *sems):
        isem, msem = sems[:NBS], sems[NBS:]
        cid = lax.axis_index("c")
        sid = lax.axis_index("s")
        zoff = pl.multiple_of(sid * ROWS_PER_SUB, 8)
        pltpu.sync_copy(zero_hbm.at[pl.ds(zoff, ROWS_PER_SUB)],
                        acc_sh.at[pl.ds(zoff, ROWS_PER_SUB)])

        base = pl.multiple_of(cid * (e_len // NC) + sid * per_w, SC_BLK)

        def start_loads(blk, b):
            off = pl.multiple_of(base + blk * SC_BLK, SC_BLK)
            pltpu.make_async_copy(dst_hbm.at[pl.ds(off, SC_BLK)],
                                  idxs.at[b], isem[b]).start()
            pltpu.make_async_copy(msg_hbm.at[pl.ds(off, SC_BLK)],
                                  bufs.at[b], msem[b]).start()

        for b in range(NBS):
            start_loads(b, b)

        plsc.subcore_barrier()

        @pl.loop(0, nblk, step=NBS)
        def _(i):
            for b in range(NBS):
                blk = i + b
                off = pl.multiple_of(base + blk * SC_BLK, SC_BLK)
                pltpu.make_async_copy(dst_hbm.at[pl.ds(off, SC_BLK)],
                                      idxs.at[b], isem[b]).wait()
                pltpu.make_async_copy(msg_hbm.at[pl.ds(off, SC_BLK)],
                                      bufs.at[b], msem[b]).wait()
                pltpu.sync_copy(bufs.at[b], acc_sh.at[idxs.at[b]], add=True)
                nxt = blk + NBS
                nxt = jnp.where(nxt >= nblk, nxt - nblk, nxt)
                start_loads(nxt, b)

        for b in range(NBS):
            off = pl.multiple_of(base + b * SC_BLK, SC_BLK)
            pltpu.make_async_copy(dst_hbm.at[pl.ds(off, SC_BLK)],
                                  idxs.at[b], isem[b]).wait()
            pltpu.make_async_copy(msg_hbm.at[pl.ds(off, SC_BLK)],
                                  bufs.at[b], msem[b]).wait()

        plsc.subcore_barrier()
        pltpu.sync_copy(acc_sh.at[pl.ds(zoff, ROWS_PER_SUB)],
                        out_hbm.at[cid].at[pl.ds(zoff, ROWS_PER_SUB)])

    return k(msg, dst, zeros)


def _msg_kernel(g_ref, ef_ref, we_ref, be_ref, o_ref):
    g = g_ref[...].astype(jnp.float32)
    o_ref[...] = jnp.maximum(
        g + _dgT(ef_ref[...], we_ref[...]) + be_ref[...], 0.0)


def _tc_msg(G, efp, We, be2d):
    e_len = G.shape[0]
    return pl.pallas_call(
        _msg_kernel,
        grid=(e_len // EBLK,),
        in_specs=[
            pl.BlockSpec((EBLK, D), lambda i: (i, 0)),
            pl.BlockSpec((EBLK, ED), lambda i: (i, 0)),
            pl.BlockSpec((D, ED), lambda i: (0, 0)),
            pl.BlockSpec((1, D), lambda i: (0, 0)),
        ],
        out_specs=pl.BlockSpec((EBLK, D), lambda i: (i, 0)),
        out_shape=jax.ShapeDtypeStruct((e_len, D), jnp.float32),
    )(G, efp, We, be2d)


def _node_kernel(em_ref, pa_ref, pb_ref, w1_ref, b1_ref, w2_ref, b2_ref,
                 gx_ref, bx_ref, wa_ref, wb_ref, xem_ref, p_ref, q_ref):
    h = (em_ref[...] + pa_ref[0, :N, :] + pa_ref[1, :N, :]
         + pb_ref[0, :N, :] + pb_ref[1, :N, :])
    h = jnp.maximum(_dgT(h, w1_ref[...]) + b1_ref[...], 0.0)
    h = _dgT(h, w2_ref[...]) + b2_ref[...]
    mu = jnp.mean(h, axis=0, keepdims=True)
    var = jnp.mean((h - mu) ** 2, axis=0, keepdims=True)
    xem = (h - mu) * lax.rsqrt(var + 1e-5) * gx_ref[...] + bx_ref[...]
    xem = jnp.maximum(xem, 0.0)
    xem_ref[...] = xem
    p_ref[:N, :] = _dgT(xem, wa_ref[...])
    q_ref[:N, :] = _dgT(xem, wb_ref[...])


def _tc_node(em, parts_a, parts_b, W1, b1r, W2, b2r, gxr, bxr, Wa, Wb):
    return pl.pallas_call(
        _node_kernel,
        out_shape=[
            jax.ShapeDtypeStruct((N, D), jnp.float32),
            jax.ShapeDtypeStruct((NPAD, D), jnp.float32),
            jax.ShapeDtypeStruct((NPAD, D), jnp.float32),
        ],
    )(em, parts_a, parts_b, W1, b1r, W2, b2r, gxr, bxr, Wa, Wb)


def _edge12_kernel(gp_ref, gq_ref, ef_ref, wc_ref, bl1_ref, wl2_ref, bl2_ref,
                   f2_ref, msum_ref, c_ref):
    pid = pl.program_id(0)

    @pl.when(pid == 0)
    def _():
        msum_ref[...] = jnp.zeros_like(msum_ref)
        c_ref[...] = jnp.zeros_like(c_ref)

    gpq = gp_ref[...].astype(jnp.float32) + gq_ref[...].astype(jnp.float32)
    f1 = jnp.maximum(
        gpq + _dgT(ef_ref[...], wc_ref[...]) + bl1_ref[...], 0.0)
    f2 = jnp.maximum(_dgT(f1, wl2_ref[...]) + bl2_ref[...], 0.0)
    f2_ref[...] = f2.astype(jnp.bfloat16)

    @pl.when(pid < N_REAL_BLOCKS)
    def _():
        msum_ref[...] += jnp.sum(f2, axis=0, keepdims=True)
        c_ref[...] += lax.dot_general(f2, f2, (((0,), (0,)), ((), ())),
                                      preferred_element_type=jnp.float32)


def _tc_edge12(GP, GQ, efp, Wc, bl1r, Wl2, bl2r):
    return pl.pallas_call(
        _edge12_kernel,
        grid=(E_PAD // EBLK,),
        in_specs=[
            pl.BlockSpec((EBLK, D), lambda i: (i, 0)),
            pl.BlockSpec((EBLK, D), lambda i: (i, 0)),
            pl.BlockSpec((EBLK, ED), lambda i: (i, 0)),
            pl.BlockSpec((D, ED), lambda i: (0, 0)),
            pl.BlockSpec((1, D), lambda i: (0, 0)),
            pl.BlockSpec((D, D), lambda i: (0, 0)),
            pl.BlockSpec((1, D), lambda i: (0, 0)),
        ],
        out_specs=[
            pl.BlockSpec((EBLK, D), lambda i: (i, 0)),
            pl.BlockSpec((1, D), lambda i: (0, 0)),
            pl.BlockSpec((D, D), lambda i: (0, 0)),
        ],
        out_shape=[
            jax.ShapeDtypeStruct((E_PAD, D), jnp.bfloat16),
            jax.ShapeDtypeStruct((1, D), jnp.float32),
            jax.ShapeDtypeStruct((D, D), jnp.float32),
        ],
    )(GP, GQ, efp, Wc, bl1r, Wl2, bl2r)


def _fold_kernel(msum_ref, c_ref, wl3_ref, bl3_ref, ge_ref, be2_ref,
                 w3s_ref, b3s_ref):
    wl3 = wl3_ref[...]
    m = msum_ref[...] / E
    bl3 = bl3_ref[...]
    wm = lax.dot_general(wl3, m, (((1,), (0,)), ((), ())),
                         preferred_element_type=jnp.float32)
    mu_e = wm + bl3
    t = lax.dot_general(wl3, c_ref[...] / E, (((1,), (0,)), ((), ())),
                        preferred_element_type=jnp.float32)
    ex2 = jnp.sum(t * wl3, axis=1, keepdims=True) + 2.0 * bl3 * wm + bl3 * bl3
    var = ex2 - mu_e * mu_e
    s = ge_ref[...] * lax.rsqrt(var + 1e-5)
    w3s_ref[...] = s * wl3
    b3s_ref[...] = s * (bl3 - mu_e) + be2_ref[...]


def _tc_fold(msum_col, C, Wl3, bl3c, gec, be2c):
    return pl.pallas_call(
        _fold_kernel,
        out_shape=[
            jax.ShapeDtypeStruct((D, D), jnp.float32),
            jax.ShapeDtypeStruct((D, 1), jnp.float32),
        ],
    )(msum_col, C, Wl3, bl3c, gec, be2c)


def _edge3_kernel(f2_ref, w3s_ref, b3s_ref, o_ref):
    o_ref[...] = jnp.maximum(
        _dgT(f2_ref[...].astype(jnp.float32), w3s_ref[...]) + b3s_ref[...],
        0.0)


def _tc_edge3(f2, W3s, b3sr):
    return pl.pallas_call(
        _edge3_kernel,
        grid=(E_PAD // EBLK,),
        in_specs=[
            pl.BlockSpec((EBLK, D), lambda i: (i, 0)),
            pl.BlockSpec((D, D), lambda i: (0, 0)),
            pl.BlockSpec((1, D), lambda i: (0, 0)),
        ],
        out_specs=pl.BlockSpec((EBLK, D), lambda i: (i, 0)),
        out_shape=jax.ShapeDtypeStruct((E_PAD, D), jnp.float32),
    )(f2, W3s, b3sr)


def kernel(em, edge_index, edge_features, W1, b1, W2, b2, We, be,
           Wl1, bl1, Wl2, bl2, Wl3, bl3, gx, bx, ge, be2):
    src = edge_index[0].astype(jnp.int32)
    dst = edge_index[1].astype(jnp.int32)
    pad = E_PAD - E
    zpad = jnp.zeros((pad,), jnp.int32)
    src_g = jnp.concatenate([src, zpad])
    dst_g = jnp.concatenate([dst, zpad])
    dst_s = jnp.concatenate([dst, jnp.full((pad,), N, jnp.int32)])
    efp = jnp.concatenate(
        [edge_features, jnp.zeros((pad, ED), jnp.float32)], axis=0)
    zeros_acc = jnp.zeros((ACC_ROWS, D), jnp.float32)

    Wa = Wl1[:, :D]
    Wb = Wl1[:, D:2 * D]
    Wc = Wl1[:, 2 * D:]

    em_p = jnp.concatenate([em, jnp.zeros((NPAD - N, D), jnp.float32)],
                           axis=0)
    H1 = E_PAD // 2
    be2d = be.reshape(1, D)
    G1 = _sc_gather(em_p, src_g[:H1])
    msg1 = _tc_msg(G1, efp[:H1], We, be2d)
    G2 = _sc_gather(em_p, src_g[H1:])
    parts1 = _sc_scatter_add(msg1, dst_s[:H1], zeros_acc)
    msg2 = _tc_msg(G2, efp[H1:], We, be2d)
    parts2 = _sc_scatter_add(msg2, dst_s[H1:], zeros_acc)

    x_em, P, Q = _tc_node(em, parts1, parts2, W1, b1.reshape(1, D), W2,
                          b2.reshape(1, D), gx.reshape(1, D),
                          bx.reshape(1, D), Wa, Wb)

    GP, GQ = _sc_gather_pq(P, Q, src_g, dst_g)

    f2, msum, C = _tc_edge12(GP, GQ, efp, Wc, bl1.reshape(1, D), Wl2,
                             bl2.reshape(1, D))

    W3s, b3s = _tc_fold(msum.reshape(D, 1), C, Wl3, bl3.reshape(D, 1),
                        ge.reshape(D, 1), be2.reshape(D, 1))
    edge_out = _tc_edge3(f2, W3s, b3s.reshape(1, D))

    return (x_em, edge_out[:E])
